# baseline (speedup 1.0000x reference)
import functools
import math

import jax
import jax.numpy as jnp
from jax import lax
from jax.experimental import pallas as pl
from jax.experimental.pallas import tpu as pltpu
from jax.experimental.pallas import tpu_sc as plsc

D_MODEL = 768
SCALE = math.sqrt(float(D_MODEL))
B = 4 * 8192

_INFO = plsc.get_sparse_core_info()
NC = _INFO.num_cores
NS = _INFO.num_subcores
L = _INFO.num_lanes
NW = NC * NS
BPW = B // NW
CH = 64
NCHUNK = BPW // CH
COLS = D_MODEL // L


NB = 2


def _emb_body(x_hbm, tab_hbm, out_hbm, idx_v,
              buf0, buf1, sg0, sg1, ss0, ss1):
    wid = lax.axis_index("s") * NC + lax.axis_index("c")
    base = wid * BPW

    pltpu.sync_copy(x_hbm.at[wid], idx_v)

    bufs = (buf0, buf1)
    gsems = (sg0, sg1)
    ssems = (ss0, ss1)
    gather = [None] * NB
    store = [None] * NB

    AHEAD = 1
    for j in range(AHEAD):
        gather[j] = pltpu.async_copy(tab_hbm.at[idx_v.at[j]], bufs[j], gsems[j])

    for j in range(NCHUNK):
        p = j % NB
        if j + AHEAD < NCHUNK:
            o = (j + AHEAD) % NB
            if store[o] is not None:
                for h in store[o]:
                    h.wait()
                store[o] = None
            gather[o] = pltpu.async_copy(
                tab_hbm.at[idx_v.at[j + AHEAD]], bufs[o], gsems[o]
            )
        gather[p].wait()
        buf = bufs[p]
        H = CH // 2

        @plsc.parallel_loop(0, H, step=1, unroll=1)
        def _scale_row_lo(r):
            for c in range(COLS):
                sl = pl.ds(c * L, L)
                buf[r, sl] = buf[r, sl] * SCALE

        st_lo = pltpu.async_copy(
            buf.at[pl.ds(0, H)],
            out_hbm.at[pl.ds(base + j * CH, H)],
            ssems[p],
        )

        @plsc.parallel_loop(H, CH, step=1, unroll=1)
        def _scale_row_hi(r):
            for c in range(COLS):
                sl = pl.ds(c * L, L)
                buf[r, sl] = buf[r, sl] * SCALE

        st_hi = pltpu.async_copy(
            buf.at[pl.ds(H, H)],
            out_hbm.at[pl.ds(base + j * CH + H, H)],
            ssems[p],
        )
        store[p] = (st_lo, st_hi)

    for pair in store:
        if pair is not None:
            for h in pair:
                h.wait()


def kernel(x, emb_weight):
    xf = x.reshape(NW, NCHUNK, CH).astype(jnp.int32)
    mesh = plsc.VectorSubcoreMesh(core_axis_name="c", subcore_axis_name="s")
    out = pl.kernel(
        _emb_body,
        out_type=jax.ShapeDtypeStruct((B, D_MODEL), jnp.float32),
        mesh=mesh,
        scratch_types=(
            [pltpu.VMEM((NCHUNK, CH), jnp.int32)]
            + [pltpu.VMEM((CH, D_MODEL), jnp.float32)] * NB
            + [pltpu.SemaphoreType.DMA] * (2 * NB)
        ),
    )(xf, emb_weight)
    return out.reshape(x.shape[0], x.shape[1], D_MODEL)


# speedup vs baseline: 1.0137x; 1.0137x over previous
import functools
import math

import jax
import jax.numpy as jnp
from jax import lax
from jax.experimental import pallas as pl
from jax.experimental.pallas import tpu as pltpu
from jax.experimental.pallas import tpu_sc as plsc

D_MODEL = 768
SCALE = math.sqrt(float(D_MODEL))
B = 4 * 8192

_INFO = plsc.get_sparse_core_info()
NC = _INFO.num_cores
NS = _INFO.num_subcores
L = _INFO.num_lanes
NW = NC * NS
BPW = B // NW
CH = 32
NCHUNK = BPW // CH
COLS = D_MODEL // L
H = CH // 2
NB = 4
AHEAD = 2


def _emb_body(x_hbm, tab_hbm, out_hbm, idx_v,
              buf0, buf1, buf2, buf3,
              sg0, sg1, sg2, sg3, ss0, ss1, ss2, ss3):
    wid = lax.axis_index("s") * NC + lax.axis_index("c")
    base = wid * BPW

    pltpu.sync_copy(x_hbm.at[wid], idx_v)

    bufs = (buf0, buf1, buf2, buf3)
    gsems = (sg0, sg1, sg2, sg3)
    ssems = (ss0, ss1, ss2, ss3)

    def issue_gather(j, b):
        pltpu.async_copy(tab_hbm.at[idx_v.at[j]], bufs[b], gsems[b])

    def wait_gather(j, b):
        pltpu.make_async_copy(tab_hbm.at[idx_v.at[j]], bufs[b], gsems[b]).wait()

    def wait_store_pair(b):
        for _ in range(2):
            pltpu.make_async_copy(
                bufs[b].at[pl.ds(0, H)],
                out_hbm.at[pl.ds(base, H)],
                ssems[b],
            ).wait()

    def scale_store(j, b):
        buf = bufs[b]

        @plsc.parallel_loop(0, H, step=1, unroll=2)
        def _scale_lo(r):
            for c in range(COLS):
                sl = pl.ds(c * L, L)
                buf[r, sl] = buf[r, sl] * SCALE

        pltpu.async_copy(
            buf.at[pl.ds(0, H)],
            out_hbm.at[pl.ds(base + j * CH, H)],
            ssems[b],
        )

        @plsc.parallel_loop(H, CH, step=1, unroll=2)
        def _scale_hi(r):
            for c in range(COLS):
                sl = pl.ds(c * L, L)
                buf[r, sl] = buf[r, sl] * SCALE

        pltpu.async_copy(
            buf.at[pl.ds(H, H)],
            out_hbm.at[pl.ds(base + j * CH + H, H)],
            ssems[b],
        )

    issue_gather(0, 0)
    issue_gather(1, 1)

    for j in (0, 1):
        issue_gather(j + AHEAD, (j + AHEAD) % NB)
        wait_gather(j, j % NB)
        scale_store(j, j % NB)

    def group_body(g, carry):
        for b4 in range(NB):
            j = 2 + g * NB + b4
            bj = (2 + b4) % NB
            bg = (2 + b4 + AHEAD) % NB
            wait_store_pair(bg)
            issue_gather(j + AHEAD, bg)
            wait_gather(j, bj)
            scale_store(j, bj)
        return carry

    lax.fori_loop(0, (NCHUNK - 4) // NB, group_body, 0)

    for j in (NCHUNK - 2, NCHUNK - 1):
        wait_gather(j, j % NB)
        scale_store(j, j % NB)

    for b in range(NB):
        wait_store_pair(b)


def kernel(x, emb_weight):
    xf = x.reshape(NW, NCHUNK, CH).astype(jnp.int32)
    mesh = plsc.VectorSubcoreMesh(core_axis_name="c", subcore_axis_name="s")
    out = pl.kernel(
        _emb_body,
        out_type=jax.ShapeDtypeStruct((B, D_MODEL), jnp.float32),
        mesh=mesh,
        scratch_types=(
            [pltpu.VMEM((NCHUNK, CH), jnp.int32)]
            + [pltpu.VMEM((CH, D_MODEL), jnp.float32)] * NB
            + [pltpu.SemaphoreType.DMA] * (2 * NB)
        ),
    )(xf, emb_weight)
    return out.reshape(x.shape[0], x.shape[1], D_MODEL)


# speedup vs baseline: 1.0951x; 1.0803x over previous
import functools
import math

import jax
import jax.numpy as jnp
from jax import lax
from jax.experimental import pallas as pl
from jax.experimental.pallas import tpu as pltpu
from jax.experimental.pallas import tpu_sc as plsc

D_MODEL = 768
SCALE = math.sqrt(float(D_MODEL))
B = 4 * 8192

_INFO = plsc.get_sparse_core_info()
NC = _INFO.num_cores
NS = _INFO.num_subcores
L = _INFO.num_lanes
NW = NC * NS
BPW = B // NW
CH = 32
NCHUNK = BPW // CH
COLS = D_MODEL // L
H = CH // 2
NB = 4
AHEAD = 2


def _emb_body(x_hbm, tab_hbm, out_hbm, idx_v,
              buf0, buf1, buf2, buf3,
              sg0, sg1, sg2, sg3, ss0, ss1, ss2, ss3):
    wid = lax.axis_index("s") * NC + lax.axis_index("c")
    base = wid * BPW

    pltpu.sync_copy(x_hbm.at[wid], idx_v)

    bufs = (buf0, buf1, buf2, buf3)
    gsems = (sg0, sg1, sg2, sg3)
    ssems = (ss0, ss1, ss2, ss3)

    def issue_gather(j, b):
        pltpu.async_copy(tab_hbm.at[idx_v.at[j]], bufs[b], gsems[b])

    def wait_gather(j, b):
        pltpu.make_async_copy(tab_hbm.at[idx_v.at[j]], bufs[b], gsems[b]).wait()

    def wait_store(b):
        pltpu.make_async_copy(
            bufs[b], out_hbm.at[pl.ds(base, CH)], ssems[b]
        ).wait()

    def scale_store(j, b):
        buf = bufs[b]

        @plsc.parallel_loop(0, CH, step=1, unroll=2)
        def _scale(r):
            for c in range(COLS):
                sl = pl.ds(c * L, L)
                buf[r, sl] = buf[r, sl] * SCALE

        pltpu.async_copy(
            buf, out_hbm.at[pl.ds(base + j * CH, CH)], ssems[b]
        )

    issue_gather(0, 0)
    issue_gather(1, 1)

    def group_body(g, carry):
        for b4 in range(NB):
            j = g * NB + b4
            bj = b4
            bg = (b4 + AHEAD) % NB

            @pl.when(j >= NB - AHEAD)
            def _():
                wait_store(bg)

            @pl.when(j + AHEAD < NCHUNK)
            def _():
                issue_gather(j + AHEAD, bg)

            wait_gather(j, bj)
            scale_store(j, bj)
        return carry

    lax.fori_loop(0, NCHUNK // NB, group_body, 0)

    for k in range(AHEAD):
        wait_store((NCHUNK - AHEAD + k) % NB)


def kernel(x, emb_weight):
    xf = x.reshape(NW, NCHUNK, CH).astype(jnp.int32)
    mesh = plsc.VectorSubcoreMesh(core_axis_name="c", subcore_axis_name="s")
    out = pl.kernel(
        _emb_body,
        out_type=jax.ShapeDtypeStruct((B, D_MODEL), jnp.float32),
        mesh=mesh,
        scratch_types=(
            [pltpu.VMEM((NCHUNK, CH), jnp.int32)]
            + [pltpu.VMEM((CH, D_MODEL), jnp.float32)] * NB
            + [pltpu.SemaphoreType.DMA] * (2 * NB)
        ),
    )(xf, emb_weight)
    return out.reshape(x.shape[0], x.shape[1], D_MODEL)


# speedup vs baseline: 1.1122x; 1.0157x over previous
import functools
import math

import jax
import jax.numpy as jnp
from jax import lax
from jax.experimental import pallas as pl
from jax.experimental.pallas import tpu as pltpu
from jax.experimental.pallas import tpu_sc as plsc

D_MODEL = 768
SCALE = math.sqrt(float(D_MODEL))
B = 4 * 8192

_INFO = plsc.get_sparse_core_info()
NC = _INFO.num_cores
NS = _INFO.num_subcores
L = _INFO.num_lanes
NW = NC * NS
BPW = B // NW
CH = 32
NCHUNK = BPW // CH
COLS = D_MODEL // L
H = CH // 2
NB = 4
AHEAD = 2


def _emb_body(x_hbm, tab_hbm, out_hbm, idx_v,
              buf0, buf1, buf2, buf3,
              sg0, sg1, sg2, sg3, ss0, ss1, ss2, ss3):
    wid = lax.axis_index("s") * NC + lax.axis_index("c")
    base = wid * BPW

    pltpu.sync_copy(x_hbm.at[wid], idx_v)

    bufs = (buf0, buf1, buf2, buf3)
    gsems = (sg0, sg1, sg2, sg3)
    ssems = (ss0, ss1, ss2, ss3)

    def issue_gather(j, b):
        pltpu.async_copy(tab_hbm.at[idx_v.at[j]], bufs[b], gsems[b])

    def wait_gather(j, b):
        pltpu.make_async_copy(tab_hbm.at[idx_v.at[j]], bufs[b], gsems[b]).wait()

    def wait_store(b):
        pltpu.make_async_copy(
            bufs[b], out_hbm.at[pl.ds(base, CH)], ssems[b]
        ).wait()

    def scale_store(j, b):
        buf = bufs[b]

        @plsc.parallel_loop(0, CH, step=1, unroll=1)
        def _scale(r):
            for c in range(COLS):
                sl = pl.ds(c * L, L)
                buf[r, sl] = buf[r, sl] * SCALE

        pltpu.async_copy(
            buf, out_hbm.at[pl.ds(base + j * CH, CH)], ssems[b]
        )

    issue_gather(0, 0)
    issue_gather(1, 1)

    def group_body(g, carry):
        for b4 in range(NB):
            j = g * NB + b4
            bj = b4
            bg = (b4 + AHEAD) % NB

            @pl.when(j >= NB - AHEAD)
            def _():
                wait_store(bg)

            @pl.when(j + AHEAD < NCHUNK)
            def _():
                issue_gather(j + AHEAD, bg)

            wait_gather(j, bj)
            scale_store(j, bj)
        return carry

    lax.fori_loop(0, NCHUNK // NB, group_body, 0)

    for k in range(AHEAD):
        wait_store((NCHUNK - AHEAD + k) % NB)


def kernel(x, emb_weight):
    xf = x.reshape(NW, NCHUNK, CH).astype(jnp.int32)
    mesh = plsc.VectorSubcoreMesh(core_axis_name="c", subcore_axis_name="s")
    out = pl.kernel(
        _emb_body,
        out_type=jax.ShapeDtypeStruct((B, D_MODEL), jnp.float32),
        mesh=mesh,
        scratch_types=(
            [pltpu.VMEM((NCHUNK, CH), jnp.int32)]
            + [pltpu.VMEM((CH, D_MODEL), jnp.float32)] * NB
            + [pltpu.SemaphoreType.DMA] * (2 * NB)
        ),
    )(xf, emb_weight)
    return out.reshape(x.shape[0], x.shape[1], D_MODEL)


# speedup vs baseline: 1.1266x; 1.0129x over previous
import functools
import math

import jax
import jax.numpy as jnp
from jax import lax
from jax.experimental import pallas as pl
from jax.experimental.pallas import tpu as pltpu
from jax.experimental.pallas import tpu_sc as plsc

D_MODEL = 768
SCALE = math.sqrt(float(D_MODEL))
B = 4 * 8192

_INFO = plsc.get_sparse_core_info()
NC = _INFO.num_cores
NS = _INFO.num_subcores
L = _INFO.num_lanes
NW = NC * NS
BPW = B // NW
CH = 32
NCHUNK = BPW // CH
COLS = D_MODEL // L
H = CH // 2
NB = 4
AHEAD = 2


def _emb_body(x_hbm, tab_hbm, out_hbm, idx_v, bigbuf, gsem, ssem):
    wid = lax.axis_index("s") * NC + lax.axis_index("c")
    base = wid * BPW

    pltpu.sync_copy(x_hbm.at[wid], idx_v)

    def issue_gather(j, b):
        pltpu.async_copy(
            tab_hbm.at[idx_v.at[j]],
            bigbuf.at[pl.ds(b * CH, CH)],
            gsem.at[b],
        )

    def wait_gather(j, b):
        pltpu.make_async_copy(
            tab_hbm.at[idx_v.at[j]],
            bigbuf.at[pl.ds(b * CH, CH)],
            gsem.at[b],
        ).wait()

    def wait_store(b):
        pltpu.make_async_copy(
            bigbuf.at[pl.ds(b * CH, CH)],
            out_hbm.at[pl.ds(base, CH)],
            ssem.at[b],
        ).wait()

    issue_gather(0, 0)
    issue_gather(1, 1)

    def chunk_body(j, carry):
        b = j % NB
        bg = (j + AHEAD) % NB

        @pl.when(j >= NB - AHEAD)
        def _():
            wait_store(bg)

        @pl.when(j + AHEAD < NCHUNK)
        def _():
            issue_gather(j + AHEAD, bg)

        wait_gather(j, b)
        rbase = b * CH

        @plsc.parallel_loop(0, CH, step=1, unroll=1)
        def _scale(r):
            for c in range(COLS):
                sl = pl.ds(c * L, L)
                bigbuf[rbase + r, sl] = bigbuf[rbase + r, sl] * SCALE

        pltpu.async_copy(
            bigbuf.at[pl.ds(rbase, CH)],
            out_hbm.at[pl.ds(base + j * CH, CH)],
            ssem.at[b],
        )
        return carry

    lax.fori_loop(0, NCHUNK, chunk_body, 0)

    for k in range(AHEAD):
        wait_store((NCHUNK - AHEAD + k) % NB)


def kernel(x, emb_weight):
    xf = x.reshape(NW, NCHUNK, CH).astype(jnp.int32)
    mesh = plsc.VectorSubcoreMesh(core_axis_name="c", subcore_axis_name="s")
    out = pl.kernel(
        _emb_body,
        out_type=jax.ShapeDtypeStruct((B, D_MODEL), jnp.float32),
        mesh=mesh,
        scratch_types=[
            pltpu.VMEM((NCHUNK, CH), jnp.int32),
            pltpu.VMEM((NB * CH, D_MODEL), jnp.float32),
            pltpu.SemaphoreType.DMA((NB,)),
            pltpu.SemaphoreType.DMA((NB,)),
        ],
    )(xf, emb_weight)
    return out.reshape(x.shape[0], x.shape[1], D_MODEL)


# speedup vs baseline: 1.1277x; 1.0010x over previous
import functools
import math

import jax
import jax.numpy as jnp
from jax import lax
from jax.experimental import pallas as pl
from jax.experimental.pallas import tpu as pltpu
from jax.experimental.pallas import tpu_sc as plsc

D_MODEL = 768
SCALE = math.sqrt(float(D_MODEL))
B = 4 * 8192

_INFO = plsc.get_sparse_core_info()
NC = _INFO.num_cores
NS = _INFO.num_subcores
L = _INFO.num_lanes
NW = NC * NS
BPW = B // NW
CH = 32
NCHUNK = BPW // CH
COLS = D_MODEL // L
H = CH // 2
NB = 4
AHEAD = 2


def _emb_body(x_hbm, tab_hbm, out_hbm, idx_v, bigbuf, gsem, ssem):
    wid = lax.axis_index("s") * NC + lax.axis_index("c")
    base = wid * BPW

    pltpu.sync_copy(x_hbm.at[wid], idx_v)

    def issue_gather(j, b):
        pltpu.async_copy(
            tab_hbm.at[idx_v.at[j]],
            bigbuf.at[pl.ds(b * CH, CH)],
            gsem.at[b],
        )

    def wait_gather(j, b):
        pltpu.make_async_copy(
            tab_hbm.at[idx_v.at[j]],
            bigbuf.at[pl.ds(b * CH, CH)],
            gsem.at[b],
        ).wait()

    def wait_store(b):
        pltpu.make_async_copy(
            bigbuf.at[pl.ds(b * CH, CH)],
            out_hbm.at[pl.ds(base, CH)],
            ssem.at[b],
        ).wait()

    issue_gather(0, 0)
    issue_gather(1, 1)

    def chunk_body(j, carry):
        b = j % NB
        bg = (j + AHEAD) % NB

        @pl.when(j >= NB - AHEAD)
        def _():
            wait_store(bg)

        @pl.when(j + AHEAD < NCHUNK)
        def _():
            issue_gather(j + AHEAD, bg)

        wait_gather(j, b)
        rbase = b * CH

        @plsc.parallel_loop(0, CH, step=1, unroll=2)
        def _scale(r):
            for c in range(COLS):
                sl = pl.ds(c * L, L)
                bigbuf[rbase + r, sl] = bigbuf[rbase + r, sl] * SCALE

        pltpu.async_copy(
            bigbuf.at[pl.ds(rbase, CH)],
            out_hbm.at[pl.ds(base + j * CH, CH)],
            ssem.at[b],
        )
        return carry

    lax.fori_loop(0, NCHUNK, chunk_body, 0)

    for k in range(AHEAD):
        wait_store((NCHUNK - AHEAD + k) % NB)


def kernel(x, emb_weight):
    xf = x.reshape(NW, NCHUNK, CH).astype(jnp.int32)
    mesh = plsc.VectorSubcoreMesh(core_axis_name="c", subcore_axis_name="s")
    out = pl.kernel(
        _emb_body,
        out_type=jax.ShapeDtypeStruct((B, D_MODEL), jnp.float32),
        mesh=mesh,
        scratch_types=[
            pltpu.VMEM((NCHUNK, CH), jnp.int32),
            pltpu.VMEM((NB * CH, D_MODEL), jnp.float32),
            pltpu.SemaphoreType.DMA((NB,)),
            pltpu.SemaphoreType.DMA((NB,)),
        ],
    )(xf, emb_weight)
    return out.reshape(x.shape[0], x.shape[1], D_MODEL)
